# baseline (speedup 1.0000x reference)
import jax
import jax.numpy as jnp
from jax.experimental import pallas as pl
from jax.experimental.pallas import tpu as pltpu


def _body(lg_ref, bx_ref, ci_ref, p_ref, lab_ref, box_ref):
    x = lg_ref[...]
    br, c = x.shape
    m = jnp.max(x, axis=-1, keepdims=True)
    ci = jnp.broadcast_to(ci_ref[...], x.shape)
    a = jnp.min(jnp.where(x == m, ci, c), axis=-1, keepdims=True)
    lab_col = jnp.where(m > 0.0, a, -1)
    rows = []
    for k in range(br // 128):
        bc = jnp.broadcast_to(lab_col[k * 128:(k + 1) * 128, :], (128, 128))
        rows.append(bc.T[0:1, :])
    lab_lane = jnp.concatenate(rows, axis=0)
    lab_ref[...] = lab_lane
    validh = (lab_lane >= 0).astype(jnp.bfloat16)
    maskf = jax.lax.dot_general(
        validh, p_ref[...], (((1,), (0,)), ((), ())),
        preferred_element_type=jnp.float32,
    )
    box_ref[...] = bx_ref[...] * maskf


def kernel(pred_logits, pred_boxes):
    B, Q, C = pred_logits.shape
    R = B * Q
    lg = pred_logits.reshape(R, C)
    bxf = pred_boxes.reshape(R * 4 // 512, 512)
    cidx = jnp.arange(C, dtype=jnp.int32).reshape(1, C)
    pmat = (jnp.arange(512, dtype=jnp.int32)[None, :] // 4
            == jnp.arange(128, dtype=jnp.int32)[:, None]).astype(jnp.bfloat16)

    BR = 8192
    BL = BR // 128
    labels, boxes = pl.pallas_call(
        _body,
        grid=(R // BR,),
        in_specs=[
            pl.BlockSpec((BR, C), lambda i: (i, 0)),
            pl.BlockSpec((BL, 512), lambda i: (i, 0)),
            pl.BlockSpec((1, C), lambda i: (0, 0)),
            pl.BlockSpec((128, 512), lambda i: (0, 0)),
        ],
        out_specs=[
            pl.BlockSpec((BL, 128), lambda i: (i, 0)),
            pl.BlockSpec((BL, 512), lambda i: (i, 0)),
        ],
        out_shape=[
            jax.ShapeDtypeStruct((R // 128, 128), jnp.int32),
            jax.ShapeDtypeStruct((R * 4 // 512, 512), jnp.float32),
        ],
    )(lg, bxf, cidx, pmat)
    labels2 = labels.reshape(R)
    num_boxes = jnp.maximum(jnp.sum(labels2 >= 0).astype(jnp.float32), 1.0)
    return labels2.reshape(B, Q), boxes.reshape(B, Q, 4), num_boxes


# speedup vs baseline: 3.1577x; 3.1577x over previous
import jax
import jax.numpy as jnp
from jax.experimental import pallas as pl
from jax.experimental.pallas import tpu as pltpu


def _body(lg_ref, bxt_ref, ci_ref, lab_ref, boxt_ref):
    x = lg_ref[...]
    br, c = x.shape
    m = jnp.max(x, axis=-1, keepdims=True)
    ci = jnp.broadcast_to(ci_ref[...], x.shape)
    a = jnp.min(jnp.where(x == m, ci, c), axis=-1, keepdims=True)
    lab_col = jnp.where(m > 0.0, a, -1)
    rows = []
    for k in range(br // 128):
        bc = jnp.broadcast_to(lab_col[k * 128:(k + 1) * 128, :], (128, 128))
        rows.append(bc.T[0:1, :])
    lab_lane = jnp.concatenate(rows, axis=0)
    valid = lab_lane >= 0
    lab_ref[...] = lab_lane
    boxt_ref[...] = jnp.where(valid[None], bxt_ref[...], 0.0)


def kernel(pred_logits, pred_boxes):
    B, Q, C = pred_logits.shape
    R = B * Q
    lg = pred_logits.reshape(R, C)
    bxt = pred_boxes.reshape(R, 4).T.reshape(4, R // 128, 128)
    cidx = jnp.arange(C, dtype=jnp.int32).reshape(1, C)

    BR = 8192
    BL = BR // 128
    labels, boxest = pl.pallas_call(
        _body,
        grid=(R // BR,),
        in_specs=[
            pl.BlockSpec((BR, C), lambda i: (i, 0)),
            pl.BlockSpec((4, BL, 128), lambda i: (0, i, 0)),
            pl.BlockSpec((1, C), lambda i: (0, 0)),
        ],
        out_specs=[
            pl.BlockSpec((BL, 128), lambda i: (i, 0)),
            pl.BlockSpec((4, BL, 128), lambda i: (0, i, 0)),
        ],
        out_shape=[
            jax.ShapeDtypeStruct((R // 128, 128), jnp.int32),
            jax.ShapeDtypeStruct((4, R // 128, 128), jnp.float32),
        ],
    )(lg, bxt, cidx)
    labels2 = labels.reshape(R)
    num_boxes = jnp.maximum(jnp.sum(labels2 >= 0).astype(jnp.float32), 1.0)
    boxes = boxest.reshape(4, R).T.reshape(B, Q, 4)
    return labels2.reshape(B, Q), boxes, num_boxes


# speedup vs baseline: 3.3631x; 1.0650x over previous
import jax
import jax.numpy as jnp
from jax.experimental import pallas as pl
from jax.experimental.pallas import tpu as pltpu


def _body(lg_ref, bxt_ref, ci_ref, lab_ref, boxt_ref):
    x = lg_ref[...]
    br, c = x.shape
    m = jnp.max(x, axis=-1, keepdims=True)
    ci = jnp.broadcast_to(ci_ref[...], x.shape)
    a = jnp.min(jnp.where(x == m, ci, c), axis=-1, keepdims=True)
    lab_col = jnp.where(m > 0.0, a, -1)
    rows = []
    for k in range(br // 128):
        bc = jnp.broadcast_to(lab_col[k * 128:(k + 1) * 128, :], (128, 128))
        rows.append(bc.T[0:1, :])
    lab_lane = jnp.concatenate(rows, axis=0)
    valid = lab_lane >= 0
    lab_ref[...] = lab_lane
    boxt_ref[...] = jnp.where(valid[None], bxt_ref[...], 0.0)


def kernel(pred_logits, pred_boxes):
    B, Q, C = pred_logits.shape
    R = B * Q
    lg = pred_logits.reshape(R, C)
    bxt = pred_boxes.reshape(R, 4).T.reshape(4, R // 128, 128)
    cidx = jnp.arange(C, dtype=jnp.int32).reshape(1, C)

    BR = 8192
    BL = BR // 128
    labels, boxest = pl.pallas_call(
        _body,
        grid=(R // BR,),
        compiler_params=pltpu.CompilerParams(
            allow_input_fusion=[False, True, False]),
        in_specs=[
            pl.BlockSpec((BR, C), lambda i: (i, 0)),
            pl.BlockSpec((4, BL, 128), lambda i: (0, i, 0)),
            pl.BlockSpec((1, C), lambda i: (0, 0)),
        ],
        out_specs=[
            pl.BlockSpec((BL, 128), lambda i: (i, 0)),
            pl.BlockSpec((4, BL, 128), lambda i: (0, i, 0)),
        ],
        out_shape=[
            jax.ShapeDtypeStruct((R // 128, 128), jnp.int32),
            jax.ShapeDtypeStruct((4, R // 128, 128), jnp.float32),
        ],
    )(lg, bxt, cidx)
    labels2 = labels.reshape(R)
    num_boxes = jnp.maximum(jnp.sum(labels2 >= 0).astype(jnp.float32), 1.0)
    boxes = boxest.reshape(4, R).T.reshape(B, Q, 4)
    return labels2.reshape(B, Q), boxes, num_boxes


# speedup vs baseline: 3.3647x; 1.0005x over previous
import jax
import jax.numpy as jnp
from jax.experimental import pallas as pl
from jax.experimental.pallas import tpu as pltpu


def _body(lg_ref, bxt_ref, ci_ref, lab_ref, boxt_ref):
    x = lg_ref[...]
    br, c = x.shape
    m = jnp.max(x, axis=-1, keepdims=True)
    ci = jnp.broadcast_to(ci_ref[...], x.shape)
    a = jnp.min(jnp.where(x == m, ci, c), axis=-1, keepdims=True)
    lab_col = jnp.where(m > 0.0, a, -1)
    rows = []
    for k in range(br // 128):
        bc = jnp.broadcast_to(lab_col[k * 128:(k + 1) * 128, :], (128, 128))
        rows.append(bc.T[0:1, :])
    lab_lane = jnp.concatenate(rows, axis=0)
    valid = lab_lane >= 0
    lab_ref[...] = lab_lane
    boxt_ref[...] = jnp.where(valid[None], bxt_ref[...], 0.0)


def kernel(pred_logits, pred_boxes):
    B, Q, C = pred_logits.shape
    R = B * Q
    lg = pred_logits.reshape(R, C)
    bxt = pred_boxes.reshape(R, 4).T.reshape(4, R // 128, 128)
    cidx = jnp.arange(C, dtype=jnp.int32).reshape(1, C)

    BR = 8192
    BL = BR // 128
    labels, boxest = pl.pallas_call(
        _body,
        grid=(R // BR,),
        compiler_params=pltpu.CompilerParams(
            allow_input_fusion=[True, True, True]),
        in_specs=[
            pl.BlockSpec((BR, C), lambda i: (i, 0)),
            pl.BlockSpec((4, BL, 128), lambda i: (0, i, 0)),
            pl.BlockSpec((1, C), lambda i: (0, 0)),
        ],
        out_specs=[
            pl.BlockSpec((BL, 128), lambda i: (i, 0)),
            pl.BlockSpec((4, BL, 128), lambda i: (0, i, 0)),
        ],
        out_shape=[
            jax.ShapeDtypeStruct((R // 128, 128), jnp.int32),
            jax.ShapeDtypeStruct((4, R // 128, 128), jnp.float32),
        ],
    )(lg, bxt, cidx)
    labels2 = labels.reshape(R)
    num_boxes = jnp.maximum(jnp.sum(labels2 >= 0).astype(jnp.float32), 1.0)
    boxes = boxest.reshape(4, R).T.reshape(B, Q, 4)
    return labels2.reshape(B, Q), boxes, num_boxes
